# baseline (speedup 1.0000x reference)
import functools
import random

import numpy as np
import jax
import jax.numpy as jnp
from jax import lax
from jax.experimental import pallas as pl
from jax.experimental.pallas import tpu as pltpu
from jax.experimental.pallas import tpu_sc as plsc


def _perm_indices(size: int) -> np.ndarray:
    h = int(np.sqrt(size))
    pad = h // 7

    def valid(idx):
        i, j = idx // h, idx % h
        return not (j < pad or i >= h - pad or j >= h - pad)

    cands = [idx for idx in range(size) if valid(idx)]
    rng = random.Random(0)
    return np.array(rng.sample(cands, len(cands)), dtype=np.int32)


_NW = 32
_L = 128
_W = 256


def kernel(x):
    B, S, D = x.shape
    perm = _perm_indices(S)
    K = perm.shape[0]
    ST = S // _L
    NP = ST // 2
    rows_d = D * NP

    k_of_token = {int(p): k for k, p in enumerate(perm)}
    ktab = []
    kmap = np.zeros((NP, _W), dtype=np.int32)
    for sp in range(NP):
        ks = [k_of_token[t] for t in range(2 * sp * _L, (2 * sp + 2) * _L)
              if t in k_of_token]
        kmap[sp, :] = 0
        for cc in range(_W):
            t = 2 * sp * _L + cc
            kmap[sp, cc] = ks.index(k_of_token[t]) if t in k_of_token else -1
        ktab.append(ks)
    gsz = max(len(ks) for ks in ktab)
    assert gsz % 32 == 0
    half = gsz // 2
    assert all(len(ks) in (gsz, half) for ks in ktab)
    nvec = gsz // 16
    junk = gsz
    obr = gsz + 8
    kmap = np.where(kmap < 0, junk, kmap).astype(np.int32)
    ktab_np = np.zeros((NP, _W), dtype=np.int32)
    for sp in range(NP):
        ks = ktab[sp]
        full = (ks * ((gsz + len(ks) - 1) // len(ks)))[:gsz]
        ktab_np[sp, :gsz] = np.asarray(full, dtype=np.int32)
    ktab_j = jnp.asarray(ktab_np)
    kmap_j = jnp.asarray(kmap)

    ntasks = D * NP
    assert ntasks % _NW == 0 and B == _L
    tpw = ntasks // _NW
    assert tpw % 2 == 0 and tpw >= 4

    mesh = plsc.VectorSubcoreMesh(core_axis_name="c", subcore_axis_name="s")

    @functools.partial(
        pl.kernel,
        mesh=mesh,
        out_type=jax.ShapeDtypeStruct((K * D, B), jnp.float32),
        scratch_types=[
            pltpu.VMEM((NP, _W), jnp.int32),
            pltpu.VMEM((NP, _W), jnp.int32),
            pltpu.VMEM((2, _L), jnp.int32),
            pltpu.VMEM((2, _L, _W), jnp.float32),
            pltpu.VMEM((2, obr, _L), jnp.float32),
            pltpu.SemaphoreType.DMA,
            pltpu.SemaphoreType.DMA,
        ],
        compiler_params=pltpu.CompilerParams(needs_layout_passes=False),
    )
    def gather_t(x_hbm, ktab_hbm, kmap_hbm, out_hbm,
                 ktab_v, kmap_v, idx_v, stage, obuf, gsem, ssem):
        w = lax.axis_index("s") * 2 + lax.axis_index("c")
        pltpu.sync_copy(ktab_hbm, ktab_v)
        pltpu.sync_copy(kmap_hbm, kmap_v)

        iota = lax.iota(jnp.int32, 16)
        diag = [(iota + t) % 16 for t in range(16)]

        def task_params(j):
            g = w * tpw + j
            return g % NP, g // NP

        def start_gather(j, bf):
            sp, d = task_params(j)
            base = d * NP + sp
            for jj in range(8):
                idx_v[bf, pl.ds(16 * jj, 16)] = (
                    iota * rows_d + (16 * jj * rows_d + base))
            pltpu.async_copy(x_hbm.at[idx_v.at[bf]], stage.at[bf], gsem)

        def transpose(j, bf):
            sp, _ = task_params(j)

            def cb_body(cb, carry):
                cols = cb * 16 + iota
                kmv = kmap_v[sp, pl.ds(cb * 16, 16)]

                def rb_body(rb, carry2):
                    r0 = rb * 16
                    for t in range(16):
                        rows = r0 + diag[t]
                        v = plsc.load_gather(stage.at[bf], [rows, cols])
                        plsc.store_scatter(obuf.at[bf], [kmv, rows], v)
                    return carry2

                return lax.fori_loop(0, 8, rb_body, carry)

            lax.fori_loop(0, 16, cb_body, 0)

        def start_scatter(j, bf):
            sp, d = task_params(j)
            shift = jnp.where(sp == NP - 1, half, 0)
            for i in range(nvec):
                src0 = 16 * i - shift if i >= nvec // 2 else 16 * i
                rows = ktab_v[sp, pl.ds(16 * i, 16)] * D + d
                pltpu.async_copy(
                    obuf.at[bf, pl.ds(src0, 16)], out_hbm.at[rows], ssem)

        def wait_gather(bf):
            pltpu.make_async_copy(
                x_hbm.at[pl.ds(0, _L)], stage.at[bf], gsem).wait()

        def wait_scatters(bf):
            pltpu.make_async_copy(
                out_hbm.at[pl.ds(0, gsz)],
                obuf.at[bf, pl.ds(0, gsz)], ssem).wait()

        start_gather(0, 0)
        start_gather(1, 1)
        wait_gather(0)
        transpose(0, 0)
        start_scatter(0, 0)
        start_gather(2 % tpw, 0)
        wait_gather(1)
        transpose(1, 1)
        start_scatter(1, 1)

        def pair_body(p, carry):
            start_gather(2 * p + 1, 1)
            wait_gather(0)
            wait_scatters(0)
            transpose(2 * p, 0)
            start_scatter(2 * p, 0)
            start_gather((2 * p + 2) % tpw, 0)
            wait_gather(1)
            wait_scatters(1)
            transpose(2 * p + 1, 1)
            start_scatter(2 * p + 1, 1)
            return carry

        lax.fori_loop(1, tpw // 2, pair_body, 0)
        wait_gather(0)
        wait_scatters(0)
        wait_scatters(1)

    xv = jnp.transpose(x, (0, 2, 1)).reshape(B * rows_d, _W)
    o_view = gather_t(xv, ktab_j, kmap_j)
    return o_view.reshape(K, D, B).transpose(2, 0, 1)


# speedup vs baseline: 1.0797x; 1.0797x over previous
import functools
import random

import numpy as np
import jax
import jax.numpy as jnp
from jax import lax
from jax.experimental import pallas as pl
from jax.experimental.pallas import tpu as pltpu
from jax.experimental.pallas import tpu_sc as plsc


def _perm_indices(size: int) -> np.ndarray:
    h = int(np.sqrt(size))
    pad = h // 7

    def valid(idx):
        i, j = idx // h, idx % h
        return not (j < pad or i >= h - pad or j >= h - pad)

    cands = [idx for idx in range(size) if valid(idx)]
    rng = random.Random(0)
    return np.array(rng.sample(cands, len(cands)), dtype=np.int32)


_NW = 32
_L = 128


def kernel(x):
    B, S, D = x.shape
    perm = _perm_indices(S)
    K = perm.shape[0]
    ST = S // _L

    sts = sorted(set(int(p) // _L for p in perm))
    nst = len(sts)
    assert sts == list(range(sts[0], sts[0] + nst))
    groups = {st: np.flatnonzero(perm // _L == st) for st in sts}
    gsz = len(groups[sts[0]])
    assert all(len(g) == gsz for g in groups.values()) and gsz % 16 == 0
    nvec = gsz // 16
    obr = gsz + 8

    ntasks = nst * D
    assert ntasks % _NW == 0 and B == _L
    tpw = ntasks // _NW
    tpp = (tpw + 3) // 4 * 4

    ktab = np.zeros((nst, _L), dtype=np.int32)
    kmap = np.zeros((nst, _L), dtype=np.int32)
    for si, st in enumerate(sts):
        ks = groups[st]
        ktab[si, :gsz] = ks
        junk = 0
        for c in range(_L):
            hits = np.flatnonzero(perm[ks] % _L == c)
            if hits.size:
                kmap[si, c] = hits[0]
            else:
                kmap[si, c] = gsz + junk % (obr - gsz)
                junk += 1
    ktab_j = jnp.asarray(ktab)
    kmap_j = jnp.asarray(kmap)

    mesh = plsc.VectorSubcoreMesh(core_axis_name="c", subcore_axis_name="s")

    @functools.partial(
        pl.kernel,
        mesh=mesh,
        out_type=jax.ShapeDtypeStruct((K * D, B), jnp.float32),
        scratch_types=[
            pltpu.VMEM((nst, _L), jnp.int32),
            pltpu.VMEM((nst, _L), jnp.int32),
            pltpu.VMEM((4, _L), jnp.int32),
            pltpu.VMEM((4, _L, _L), jnp.float32),
            pltpu.VMEM((2, obr, _L), jnp.float32),
            pltpu.SemaphoreType.DMA,
            pltpu.SemaphoreType.DMA,
        ],
        compiler_params=pltpu.CompilerParams(needs_layout_passes=False),
    )
    def gather_t(x_hbm, ktab_hbm, kmap_hbm, out_hbm,
                 ktab_v, kmap_v, idx_v, stage, obuf, gsem, ssem):
        w = lax.axis_index("s") * 2 + lax.axis_index("c")
        pltpu.sync_copy(ktab_hbm, ktab_v)
        pltpu.sync_copy(kmap_hbm, kmap_v)

        iota = lax.iota(jnp.int32, 16)
        diag = [(iota + t) % 16 for t in range(16)]
        row_step = D * ST

        def task_params(j):
            j = jnp.where(j >= 2 * tpp, j - 2 * tpp, j)
            j = jnp.where(j >= tpp, j - tpp, j)
            j = jnp.where(j >= tpw, j - tpw, j)
            g = w * tpw + j
            return g // D, g % D

        def start_gather(j, bf):
            si, d = task_params(j)
            base = d * ST + (si + sts[0])
            for jj in range(8):
                idx_v[bf, pl.ds(16 * jj, 16)] = (
                    iota * row_step + (16 * jj * row_step + base))
            pltpu.async_copy(x_hbm.at[idx_v.at[bf]], stage.at[bf], gsem)

        def transpose(j, bf, bf2):
            si, _ = task_params(j)

            def cb_body(cb, carry):
                cols = cb * 16 + iota
                kmv = kmap_v[si, pl.ds(cb * 16, 16)]

                def rb_body(rb, carry2):
                    r0 = rb * 16
                    for t in range(16):
                        rows = r0 + diag[t]
                        v = plsc.load_gather(stage.at[bf], [rows, cols])
                        plsc.store_scatter(obuf.at[bf2], [kmv, rows], v)
                    return carry2

                return lax.fori_loop(0, 8, rb_body, carry)

            lax.fori_loop(0, 8, cb_body, 0)

        def start_scatter(j, bf2):
            si, d = task_params(j)
            for i in range(nvec):
                rows = ktab_v[si, pl.ds(16 * i, 16)] * D + d
                pltpu.async_copy(
                    obuf.at[bf2, pl.ds(16 * i, 16)], out_hbm.at[rows], ssem)

        def wait_gather(bf):
            pltpu.make_async_copy(
                x_hbm.at[pl.ds(0, _L)], stage.at[bf], gsem).wait()

        def wait_scatters(bf2):
            pltpu.make_async_copy(
                out_hbm.at[pl.ds(0, gsz)],
                obuf.at[bf2, pl.ds(0, gsz)], ssem).wait()

        def step(j, m, wait_s):
            wait_gather(m)
            if wait_s:
                wait_scatters(m % 2)
            transpose(j, m, m % 2)
            start_scatter(j, m % 2)
            start_gather(j + 4, m)

        for m in range(4):
            start_gather(m, m)
        for m in range(4):
            step(m, m, wait_s=(m >= 2))

        def quad_body(q, carry):
            for m in range(4):
                step(4 * q + m, m, wait_s=True)
            return carry

        lax.fori_loop(1, tpp // 4, quad_body, 0)
        for m in range(4):
            wait_gather(m)
        wait_scatters(0)
        wait_scatters(1)

    xv = jnp.transpose(x, (0, 2, 1)).reshape(B * D * ST, _L)
    o_view = gather_t(xv, ktab_j, kmap_j)
    return o_view.reshape(K, D, B).transpose(2, 0, 1)


# speedup vs baseline: 1.5830x; 1.4662x over previous
import functools
import random

import numpy as np
import jax
import jax.numpy as jnp
from jax import lax
from jax.experimental import pallas as pl
from jax.experimental.pallas import tpu as pltpu
from jax.experimental.pallas import tpu_sc as plsc


def _perm_indices(size: int) -> np.ndarray:
    h = int(np.sqrt(size))
    pad = h // 7

    def valid(idx):
        i, j = idx // h, idx % h
        return not (j < pad or i >= h - pad or j >= h - pad)

    cands = [idx for idx in range(size) if valid(idx)]
    rng = random.Random(0)
    return np.array(rng.sample(cands, len(cands)), dtype=np.int32)


_NW = 32
_L = 128


def kernel(x):
    B, S, D = x.shape
    perm = _perm_indices(S)
    K = perm.shape[0]
    ST = S // _L

    sts = sorted(set(int(p) // _L for p in perm))
    nst = len(sts)
    assert sts == list(range(sts[0], sts[0] + nst))
    groups = {st: np.flatnonzero(perm // _L == st) for st in sts}
    gsz = len(groups[sts[0]])
    assert all(len(g) == gsz for g in groups.values()) and gsz % 16 == 0
    nvec = gsz // 16
    obr = gsz + 8

    ntasks = nst * D
    assert ntasks % _NW == 0 and B == _L
    tpw = ntasks // _NW
    tpp = (tpw + 3) // 4 * 4

    ktab = np.zeros((nst, _L), dtype=np.int32)
    kmap = np.zeros((nst, _L), dtype=np.int32)
    for si, st in enumerate(sts):
        ks = groups[st]
        ktab[si, :gsz] = ks
        junk = 0
        for c in range(_L):
            hits = np.flatnonzero(perm[ks] % _L == c)
            if hits.size:
                kmap[si, c] = hits[0]
            else:
                kmap[si, c] = gsz + junk % (obr - gsz)
                junk += 1
    ktab_j = jnp.asarray(ktab)
    kmap_j = jnp.asarray(kmap)

    mesh = plsc.VectorSubcoreMesh(core_axis_name="c", subcore_axis_name="s")

    @functools.partial(
        pl.kernel,
        mesh=mesh,
        out_type=jax.ShapeDtypeStruct((K * D, B), jnp.float32),
        scratch_types=[
            pltpu.VMEM((nst, _L), jnp.int32),
            pltpu.VMEM((nst, _L), jnp.int32),
            pltpu.VMEM((4, _L), jnp.int32),
            pltpu.VMEM((4, _L, _L), jnp.float32),
            pltpu.VMEM((2, obr, _L), jnp.float32),
            pltpu.SemaphoreType.DMA,
            pltpu.SemaphoreType.DMA,
        ],
        compiler_params=pltpu.CompilerParams(needs_layout_passes=False),
    )
    def gather_t(x_hbm, ktab_hbm, kmap_hbm, out_hbm,
                 ktab_v, kmap_v, idx_v, stage, obuf, gsem, ssem):
        w = lax.axis_index("s") * 2 + lax.axis_index("c")
        pltpu.sync_copy(ktab_hbm, ktab_v)
        pltpu.sync_copy(kmap_hbm, kmap_v)

        iota = lax.iota(jnp.int32, 16)
        diag = [(iota + t) % 16 for t in range(16)]
        row_step = D * ST

        def task_params(j):
            j = jnp.where(j >= 2 * tpp, j - 2 * tpp, j)
            j = jnp.where(j >= tpp, j - tpp, j)
            j = jnp.where(j >= tpw, j - tpw, j)
            g = w * tpw + j
            return g // D, g % D

        def start_gather(j, bf):
            si, d = task_params(j)
            pltpu.async_copy(
                x_hbm.at[:, d, pl.ds((si + sts[0]) * _L, _L)],
                stage.at[bf], gsem)

        def transpose(j, bf, bf2):
            si, _ = task_params(j)

            def cb_body(cb, carry):
                cols = cb * 16 + iota
                kmv = kmap_v[si, pl.ds(cb * 16, 16)]

                def rb_body(rb, carry2):
                    r0 = rb * 16
                    for t in range(16):
                        rows = r0 + diag[t]
                        v = plsc.load_gather(stage.at[bf], [rows, cols])
                        plsc.store_scatter(obuf.at[bf2], [kmv, rows], v)
                    return carry2

                return lax.fori_loop(0, 8, rb_body, carry)

            lax.fori_loop(0, 8, cb_body, 0)

        def start_scatter(j, bf2):
            si, d = task_params(j)
            for i in range(nvec):
                rows = ktab_v[si, pl.ds(16 * i, 16)] * D + d
                pltpu.async_copy(
                    obuf.at[bf2, pl.ds(16 * i, 16)], out_hbm.at[rows], ssem)

        def wait_gather(bf):
            pltpu.make_async_copy(
                x_hbm.at[:, 0, pl.ds(0, _L)], stage.at[bf], gsem).wait()

        def wait_scatters(bf2):
            pltpu.make_async_copy(
                out_hbm.at[pl.ds(0, gsz)],
                obuf.at[bf2, pl.ds(0, gsz)], ssem).wait()

        def step(j, m, wait_s):
            wait_gather(m)
            if wait_s:
                wait_scatters(m % 2)
            transpose(j, m, m % 2)
            start_scatter(j, m % 2)
            start_gather(j + 4, m)

        for m in range(4):
            start_gather(m, m)
        for m in range(4):
            step(m, m, wait_s=(m >= 2))

        def quad_body(q, carry):
            for m in range(4):
                step(4 * q + m, m, wait_s=True)
            return carry

        lax.fori_loop(1, tpp // 4, quad_body, 0)
        for m in range(4):
            wait_gather(m)
        wait_scatters(0)
        wait_scatters(1)

    xv = jnp.transpose(x, (0, 2, 1))
    o_view = gather_t(xv, ktab_j, kmap_j)
    return o_view.reshape(K, D, B).transpose(2, 0, 1)


# speedup vs baseline: 1.6667x; 1.0529x over previous
import functools
import random

import numpy as np
import jax
import jax.numpy as jnp
from jax import lax
from jax.experimental import pallas as pl
from jax.experimental.pallas import tpu as pltpu
from jax.experimental.pallas import tpu_sc as plsc


def _perm_indices(size: int) -> np.ndarray:
    h = int(np.sqrt(size))
    pad = h // 7

    def valid(idx):
        i, j = idx // h, idx % h
        return not (j < pad or i >= h - pad or j >= h - pad)

    cands = [idx for idx in range(size) if valid(idx)]
    rng = random.Random(0)
    return np.array(rng.sample(cands, len(cands)), dtype=np.int32)


_NW = 32
_L = 128


def kernel(x):
    B, S, D = x.shape
    perm = _perm_indices(S)
    K = perm.shape[0]
    ST = S // _L

    sts = sorted(set(int(p) // _L for p in perm))
    nst = len(sts)
    assert sts == list(range(sts[0], sts[0] + nst))
    groups = {st: np.flatnonzero(perm // _L == st) for st in sts}
    gsz = len(groups[sts[0]])
    assert all(len(g) == gsz for g in groups.values()) and gsz % 16 == 0
    nvec = gsz // 16
    obr = gsz + 8

    ntasks = nst * D
    assert ntasks % _NW == 0 and B == _L
    tpw = ntasks // _NW
    tpp = (tpw + 3) // 4 * 4

    ktab = np.zeros((nst, _L), dtype=np.int32)
    kmap = np.zeros((nst, _L), dtype=np.int32)
    for si, st in enumerate(sts):
        ks = groups[st]
        ktab[si, :gsz] = ks
        junk = 0
        for c in range(_L):
            hits = np.flatnonzero(perm[ks] % _L == c)
            if hits.size:
                kmap[si, c] = hits[0]
            else:
                kmap[si, c] = gsz + junk % (obr - gsz)
                junk += 1
    ktab_j = jnp.asarray(ktab)
    kmap_j = jnp.asarray(kmap)

    mesh = plsc.VectorSubcoreMesh(core_axis_name="c", subcore_axis_name="s")

    @functools.partial(
        pl.kernel,
        mesh=mesh,
        out_type=jax.ShapeDtypeStruct((K * D, B), jnp.float32),
        scratch_types=[
            pltpu.VMEM((nst, _L), jnp.int32),
            pltpu.VMEM((nst, _L), jnp.int32),
            pltpu.VMEM((2, _L, _L), jnp.float32),
            pltpu.VMEM((2, obr, _L), jnp.float32),
            pltpu.SemaphoreType.DMA,
            pltpu.SemaphoreType.DMA,
        ],
        compiler_params=pltpu.CompilerParams(needs_layout_passes=False),
    )
    def gather_t(x_hbm, ktab_hbm, kmap_hbm, out_hbm,
                 ktab_v, kmap_v, stage, obuf, gsem, ssem):
        w = lax.axis_index("s") * 2 + lax.axis_index("c")
        pltpu.sync_copy(ktab_hbm, ktab_v)
        pltpu.sync_copy(kmap_hbm, kmap_v)

        iota = lax.iota(jnp.int32, 16)
        diag = [(iota + t) % 16 for t in range(16)]
        row_step = D * ST

        def task_params(j):
            j = jnp.where(j >= tpw, j - tpw, j)
            g = w * tpw + j
            return g // D, g % D

        def start_gather(j, bf):
            si, d = task_params(j)
            pltpu.async_copy(
                x_hbm.at[:, d, pl.ds((si + sts[0]) * _L, _L)],
                stage.at[bf], gsem)

        def transpose(j, bf, bf2):
            si, _ = task_params(j)

            def cb_body(cb, carry):
                cols = cb * 16 + iota
                kmv = kmap_v[si, pl.ds(cb * 16, 16)]

                def rb_body(rb, carry2):
                    r0 = rb * 16
                    for t in range(16):
                        rows = r0 + diag[t]
                        v = plsc.load_gather(stage.at[bf], [rows, cols])
                        plsc.store_scatter(obuf.at[bf2], [kmv, rows], v)
                    return carry2

                return lax.fori_loop(0, 8, rb_body, carry)

            lax.fori_loop(0, 8, cb_body, 0)

        def start_scatter(j, bf2):
            si, d = task_params(j)
            for i in range(nvec):
                rows = ktab_v[si, pl.ds(16 * i, 16)] * D + d
                pltpu.async_copy(
                    obuf.at[bf2, pl.ds(16 * i, 16)], out_hbm.at[rows], ssem)

        def wait_gather(bf):
            pltpu.make_async_copy(
                x_hbm.at[:, 0, pl.ds(0, _L)], stage.at[bf], gsem).wait()

        def wait_scatters(bf):
            pltpu.make_async_copy(
                out_hbm.at[pl.ds(0, gsz)],
                obuf.at[bf, pl.ds(0, gsz)], ssem).wait()

        def step(j, m, wait_s):
            wait_gather(m)
            if wait_s:
                wait_scatters(m)
            transpose(j, m, m)
            start_scatter(j, m)
            start_gather(j + 2, m)

        start_gather(0, 0)
        start_gather(1, 1)
        step(0, 0, wait_s=False)
        step(1, 1, wait_s=False)

        def pair_body(p, carry):
            step(2 * p, 0, wait_s=True)
            step(2 * p + 1, 1, wait_s=True)
            return carry

        lax.fori_loop(1, tpw // 2, pair_body, 0)
        wait_gather(0)
        wait_gather(1)
        wait_scatters(0)
        wait_scatters(1)

    xv = jnp.transpose(x, (0, 2, 1))
    o_view = gather_t(xv, ktab_j, kmap_j)
    return o_view.reshape(K, D, B).transpose(2, 0, 1)
